# baseline (speedup 1.0000x reference)
import functools

import jax
import jax.numpy as jnp
from jax import lax
from jax.experimental import pallas as pl
from jax.experimental.pallas import tpu as pltpu
from jax.experimental.pallas import tpu_sc as plsc

DIM = 128
BATCH = 16384
_NC = 2
_NS = 16
_NW = _NC * _NS

_TC_BATCH = 512
_SC_BATCH = BATCH - _TC_BATCH
_B_PER_W = _SC_BATCH // _NW

_CHUNKS = 2
_ROWS = _B_PER_W // _CHUNKS
_VOCAB = 1000
_VOCAB_PAD = 1024


@functools.partial(
    pl.kernel,
    mesh=plsc.VectorSubcoreMesh(core_axis_name="c", subcore_axis_name="s"),
    out_type=jax.ShapeDtypeStruct((BATCH, DIM), jnp.float32),
    scratch_types=[
        pltpu.VMEM((_B_PER_W,), jnp.int32),
        pltpu.VMEM((_CHUNKS, _ROWS, DIM), jnp.float32),
        pltpu.VMEM_SHARED((_VOCAB, DIM), jnp.float32),
        pltpu.SemaphoreType.DMA,
        pltpu.SemaphoreType.DMA,
    ],
)
def _gather_rows(idx_hbm, table_hbm, out_hbm, idx_v, rows_v, table_s, gsem, wsem):
    sid = lax.axis_index("s")
    wid = sid * _NC + lax.axis_index("c")
    base = wid * _B_PER_W
    icp = pltpu.async_copy(idx_hbm.at[pl.ds(base, _B_PER_W)], idx_v, wsem)
    for t, (off, ln) in enumerate(((0, 256), (256, 256), (512, 256), (768, 232))):
        @pl.when(sid == t)
        def _(off=off, ln=ln):
            pltpu.sync_copy(table_hbm.at[pl.ds(off, ln)], table_s.at[pl.ds(off, ln)])

    icp.wait()
    plsc.subcore_barrier()
    gathers = [
        pltpu.async_copy(
            table_s.at[idx_v.at[pl.ds(j * _ROWS, _ROWS)]], rows_v.at[j], gsem
        )
        for j in range(_CHUNKS)
    ]
    writes = []
    for j in range(_CHUNKS):
        gathers[j].wait()
        writes.append(
            pltpu.async_copy(
                rows_v.at[j], out_hbm.at[pl.ds(base + j * _ROWS, _ROWS)], wsem
            )
        )
    for w in writes:
        w.wait()


def _onehot_block(idx_ref, table_ref, out_ref):
    idx = idx_ref[0]
    onehot = (
        idx[:, None] == lax.broadcasted_iota(jnp.int32, (idx.shape[0], _VOCAB_PAD), 1)
    ).astype(jnp.float32)
    out_ref[...] = jnp.dot(
        onehot, table_ref[...], preferred_element_type=jnp.float32
    )


def _tc_tail(idx_tail, table_pad):
    return pl.pallas_call(
        _onehot_block,
        grid=(1,),
        in_specs=[
            pl.BlockSpec((1, _TC_BATCH), lambda i: (0, 0)),
            pl.BlockSpec((_VOCAB_PAD, DIM), lambda i: (0, 0)),
        ],
        out_specs=pl.BlockSpec((_TC_BATCH, DIM), lambda i: (0, 0)),
        out_shape=jax.ShapeDtypeStruct((_TC_BATCH, DIM), jnp.float32),
    )(idx_tail.reshape(1, _TC_BATCH), table_pad)


def kernel(x, table):
    x = x.astype(jnp.int32)
    table_pad = jnp.pad(table, ((0, _VOCAB_PAD - _VOCAB), (0, 0)))
    tc_part = _tc_tail(x[_SC_BATCH:], table_pad)
    out = _gather_rows(x, table)
    out = lax.dynamic_update_slice(out, tc_part, (_SC_BATCH, 0))
    return out.reshape(BATCH, 1, DIM)


# speedup vs baseline: 1.0984x; 1.0984x over previous
import functools

import jax
import jax.numpy as jnp
from jax import lax
from jax.experimental import pallas as pl
from jax.experimental.pallas import tpu as pltpu
from jax.experimental.pallas import tpu_sc as plsc

DIM = 128
BATCH = 16384
_NC = 2
_NS = 16
_NW = _NC * _NS
_B_PER_W = BATCH // _NW


_CHUNKS = 8
_ROWS = _B_PER_W // _CHUNKS
_VOCAB = 1000


@functools.partial(
    pl.kernel,
    mesh=plsc.VectorSubcoreMesh(core_axis_name="c", subcore_axis_name="s"),
    out_type=jax.ShapeDtypeStruct((BATCH, DIM), jnp.float32),
    scratch_types=[
        pltpu.VMEM((_B_PER_W,), jnp.int32),
        pltpu.VMEM((_CHUNKS, _ROWS, DIM), jnp.float32),
        pltpu.VMEM_SHARED((_VOCAB, DIM), jnp.float32),
        pltpu.SemaphoreType.DMA,
        pltpu.SemaphoreType.DMA,
    ],
)
def _gather_rows(idx_hbm, table_hbm, out_hbm, idx_v, rows_v, table_s, gsem, wsem):
    sid = lax.axis_index("s")
    wid = sid * _NC + lax.axis_index("c")
    base = wid * _B_PER_W
    icp = pltpu.async_copy(idx_hbm.at[pl.ds(base, _B_PER_W)], idx_v, wsem)
    _PIECES = tuple((t * 128, 128 if t < 7 else 104) for t in range(8))
    for t, (off, ln) in enumerate(_PIECES):
        @pl.when(sid == t)
        def _(off=off, ln=ln):
            pltpu.sync_copy(table_hbm.at[pl.ds(off, ln)], table_s.at[pl.ds(off, ln)])

    icp.wait()
    plsc.subcore_barrier()
    gathers = [
        pltpu.async_copy(
            table_s.at[idx_v.at[pl.ds(j * _ROWS, _ROWS)]], rows_v.at[j], gsem
        )
        for j in range(_CHUNKS)
    ]
    writes = []
    for j in range(_CHUNKS):
        gathers[j].wait()
        writes.append(
            pltpu.async_copy(
                rows_v.at[j], out_hbm.at[pl.ds(base + j * _ROWS, _ROWS)], wsem
            )
        )
    for w in writes:
        w.wait()


def kernel(x, table):
    out = _gather_rows(x.astype(jnp.int32), table)
    return out.reshape(BATCH, 1, DIM)
